# baseline (speedup 1.0000x reference)
import functools

import numpy as np
import jax
import jax.numpy as jnp
from jax.experimental import pallas as pl
from jax.experimental.pallas import tpu as pltpu

_WINDOW = 11
_SIGMA = 1.5
_DATA_RANGE = 255.0
_K1 = 0.01
_K2 = 0.03


def _gauss_taps(window_size: int, sigma: float) -> np.ndarray:
    x = np.arange(window_size, dtype=np.float64) - window_size // 2
    g = np.exp(-(x * x) / (2.0 * sigma * sigma))
    return (g / g.sum()).astype(np.float32)


def _row_blur_matrix(n: int, taps: np.ndarray) -> np.ndarray:
    pad = taps.shape[0] // 2
    m = np.zeros((n, n), dtype=np.float32)
    for t, w in enumerate(taps):
        d = pad - t
        if abs(d) < n:
            m += w * np.eye(n, k=d, dtype=np.float32)
    return m


def _ssim_batch_kernel(x1_ref, x2_ref, aw_ref, av_ref, out_ref,
                       *, P, H, W, C1, C2):
    total = jnp.float32(0.0)
    for p in range(P):
        x1 = x1_ref[p]
        x2 = x2_ref[p]
        blur = []
        for v in (x1, x2, x1 * x1 + x2 * x2, x1 * x2):
            u = jnp.dot(v.astype(jnp.bfloat16), aw_ref[...],
                        preferred_element_type=jnp.float32)
            blur.append(jnp.dot(av_ref[...], u.astype(jnp.bfloat16),
                                preferred_element_type=jnp.float32))
        mu1, mu2, e_ss, e12 = blur

        mu_ss = mu1 * mu1 + mu2 * mu2
        mu12 = mu1 * mu2
        num = (2.0 * mu12 + C1) * (2.0 * (e12 - mu12) + C2)
        den = (mu_ss + C1) * ((e_ss - mu_ss) + C2)
        r = pl.reciprocal(den, approx=True)
        r = r * (2.0 - den * r)
        total = total + jnp.sum(num * r)
    out_ref[...] = jnp.full(out_ref.shape, total, out_ref.dtype)


def _ssim_loss(img1, img2):
    N, n_ch, H, W = img1.shape
    nplanes = N * n_ch

    P = next(p for p in (8, 6, 4, 3, 2, 1) if nplanes % p == 0)
    S = nplanes // P

    taps = _gauss_taps(_WINDOW, _SIGMA)
    aw = jnp.asarray(_row_blur_matrix(W, taps)).astype(jnp.bfloat16)
    av = jnp.asarray(_row_blur_matrix(H, taps).T).astype(jnp.bfloat16)

    C1 = float((_K1 * _DATA_RANGE) ** 2)
    C2 = float((_K2 * _DATA_RANGE) ** 2)

    x1 = img1.astype(jnp.float32).reshape(nplanes, H, W)
    x2 = img2.astype(jnp.float32).reshape(nplanes, H, W)

    body = functools.partial(_ssim_batch_kernel, P=P, H=H, W=W, C1=C1, C2=C2)
    partials = pl.pallas_call(
        body,
        out_shape=jax.ShapeDtypeStruct((S, 8, 128), jnp.float32),
        grid=(S,),
        in_specs=[
            pl.BlockSpec((P, H, W), lambda i: (i, 0, 0)),
            pl.BlockSpec((P, H, W), lambda i: (i, 0, 0)),
            pl.BlockSpec((W, W), lambda i: (0, 0)),
            pl.BlockSpec((H, H), lambda i: (0, 0)),
        ],
        out_specs=pl.BlockSpec((1, 8, 128), lambda i: (i, 0, 0)),
        compiler_params=pltpu.CompilerParams(
            dimension_semantics=("parallel",)),
    )(x1, x2, aw, av)

    mean_ssim = jnp.sum(partials[:, 0, 0]) / float(nplanes * H * W)
    return 1.0 - mean_ssim


def kernel(img1, img2):
    return _ssim_loss(img1, img2)


# speedup vs baseline: 1.1299x; 1.1299x over previous
import functools

import numpy as np
import jax
import jax.numpy as jnp
from jax.experimental import pallas as pl
from jax.experimental.pallas import tpu as pltpu

_WINDOW = 11
_SIGMA = 1.5
_DATA_RANGE = 255.0
_K1 = 0.01
_K2 = 0.03


def _gauss_taps(window_size: int, sigma: float) -> np.ndarray:
    x = np.arange(window_size, dtype=np.float64) - window_size // 2
    g = np.exp(-(x * x) / (2.0 * sigma * sigma))
    return (g / g.sum()).astype(np.float32)


def _row_blur_matrix(n: int, taps: np.ndarray) -> np.ndarray:
    pad = taps.shape[0] // 2
    m = np.zeros((n, n), dtype=np.float32)
    for t, w in enumerate(taps):
        d = pad - t
        if abs(d) < n:
            m += w * np.eye(n, k=d, dtype=np.float32)
    return m


def _ssim_batch_kernel(x1_ref, x2_ref, aw_ref, av_ref, out_ref,
                       *, P, H, W, C1, C2):
    total = jnp.float32(0.0)
    for p in range(P):
        x1 = x1_ref[p].astype(jnp.bfloat16)
        x2 = x2_ref[p].astype(jnp.bfloat16)
        stack = jnp.concatenate(
            (x1, x2, x1 * x1 + x2 * x2, x1 * x2), axis=0)
        u = jnp.dot(stack, aw_ref[...],
                    preferred_element_type=jnp.float32).astype(jnp.bfloat16)
        blur = [jnp.dot(av_ref[...], u[s * H:(s + 1) * H],
                        preferred_element_type=jnp.float32).astype(jnp.bfloat16)
                for s in range(4)]
        mu1, mu2, e_ss, e12 = blur

        one = jnp.bfloat16(1.0)
        mu_ss = mu1 * mu1 + mu2 * mu2
        mu12 = mu1 * mu2
        num = (mu12 + mu12 + jnp.bfloat16(C1)) * \
            ((e12 - mu12) * jnp.bfloat16(2.0) + jnp.bfloat16(C2))
        den = (mu_ss + jnp.bfloat16(C1)) * ((e_ss - mu_ss) + jnp.bfloat16(C2))
        rs = jax.lax.rsqrt(den)
        total = total + jnp.sum(num * rs * rs, dtype=jnp.float32)
    out_ref[...] = jnp.full(out_ref.shape, total, out_ref.dtype)


def _ssim_loss(img1, img2):
    N, n_ch, H, W = img1.shape
    nplanes = N * n_ch

    P = next(p for p in (8, 6, 4, 3, 2, 1) if nplanes % p == 0)
    S = nplanes // P

    taps = _gauss_taps(_WINDOW, _SIGMA)
    aw = jnp.asarray(_row_blur_matrix(W, taps)).astype(jnp.bfloat16)
    av = jnp.asarray(_row_blur_matrix(H, taps).T).astype(jnp.bfloat16)

    C1 = float((_K1 * _DATA_RANGE) ** 2)
    C2 = float((_K2 * _DATA_RANGE) ** 2)

    x1 = img1.astype(jnp.float32).reshape(nplanes, H, W)
    x2 = img2.astype(jnp.float32).reshape(nplanes, H, W)

    body = functools.partial(_ssim_batch_kernel, P=P, H=H, W=W, C1=C1, C2=C2)
    partials = pl.pallas_call(
        body,
        out_shape=jax.ShapeDtypeStruct((S, 8, 128), jnp.float32),
        grid=(S,),
        in_specs=[
            pl.BlockSpec((P, H, W), lambda i: (i, 0, 0)),
            pl.BlockSpec((P, H, W), lambda i: (i, 0, 0)),
            pl.BlockSpec((W, W), lambda i: (0, 0)),
            pl.BlockSpec((H, H), lambda i: (0, 0)),
        ],
        out_specs=pl.BlockSpec((1, 8, 128), lambda i: (i, 0, 0)),
        compiler_params=pltpu.CompilerParams(
            dimension_semantics=("parallel",)),
    )(x1, x2, aw, av)

    mean_ssim = jnp.sum(partials[:, 0, 0]) / float(nplanes * H * W)
    return 1.0 - mean_ssim


def kernel(img1, img2):
    return _ssim_loss(img1, img2)


# speedup vs baseline: 1.1435x; 1.0120x over previous
import functools

import numpy as np
import jax
import jax.numpy as jnp
from jax.experimental import pallas as pl
from jax.experimental.pallas import tpu as pltpu

_WINDOW = 11
_SIGMA = 1.5
_DATA_RANGE = 255.0
_K1 = 0.01
_K2 = 0.03


def _gauss_taps(window_size: int, sigma: float) -> np.ndarray:
    x = np.arange(window_size, dtype=np.float64) - window_size // 2
    g = np.exp(-(x * x) / (2.0 * sigma * sigma))
    return (g / g.sum()).astype(np.float32)


def _row_blur_matrix(n: int, taps: np.ndarray) -> np.ndarray:
    pad = taps.shape[0] // 2
    m = np.zeros((n, n), dtype=np.float32)
    for t, w in enumerate(taps):
        d = pad - t
        if abs(d) < n:
            m += w * np.eye(n, k=d, dtype=np.float32)
    return m


def _ssim_batch_kernel(x1_ref, x2_ref, aw_ref, av_ref, out_ref,
                       *, P, H, W, C1, C2):
    total = jnp.float32(0.0)
    for p in range(P):
        x1 = x1_ref[p].astype(jnp.bfloat16)
        x2 = x2_ref[p].astype(jnp.bfloat16)
        stack = jnp.concatenate(
            (x1, x2, x1 * x1 + x2 * x2, x1 * x2), axis=0)
        u = jnp.dot(stack, aw_ref[...],
                    preferred_element_type=jnp.float32).astype(jnp.bfloat16)
        blur = [jnp.dot(av_ref[...], u[s * H:(s + 1) * H],
                        preferred_element_type=jnp.float32).astype(jnp.bfloat16)
                for s in range(4)]
        mu1, mu2, e_ss, e12 = blur

        one = jnp.bfloat16(1.0)
        mu_ss = mu1 * mu1 + mu2 * mu2
        mu12 = mu1 * mu2
        num = (mu12 + mu12 + jnp.bfloat16(C1)) * \
            ((e12 - mu12) * jnp.bfloat16(2.0) + jnp.bfloat16(C2))
        den = (mu_ss + jnp.bfloat16(C1)) * ((e_ss - mu_ss) + jnp.bfloat16(C2))
        rs = jax.lax.rsqrt(den)
        total = total + jnp.sum(num * rs * rs, dtype=jnp.float32)
    out_ref[...] = jnp.full(out_ref.shape, total, out_ref.dtype)


def _ssim_loss(img1, img2):
    N, n_ch, H, W = img1.shape
    nplanes = N * n_ch

    P = next(p for p in (12, 8, 6, 4, 3, 2, 1) if nplanes % p == 0)
    S = nplanes // P

    taps = _gauss_taps(_WINDOW, _SIGMA)
    aw = jnp.asarray(_row_blur_matrix(W, taps)).astype(jnp.bfloat16)
    av = jnp.asarray(_row_blur_matrix(H, taps).T).astype(jnp.bfloat16)

    C1 = float((_K1 * _DATA_RANGE) ** 2)
    C2 = float((_K2 * _DATA_RANGE) ** 2)

    x1 = img1.astype(jnp.float32).reshape(nplanes, H, W)
    x2 = img2.astype(jnp.float32).reshape(nplanes, H, W)

    body = functools.partial(_ssim_batch_kernel, P=P, H=H, W=W, C1=C1, C2=C2)
    partials = pl.pallas_call(
        body,
        out_shape=jax.ShapeDtypeStruct((S, 8, 128), jnp.float32),
        grid=(S,),
        in_specs=[
            pl.BlockSpec((P, H, W), lambda i: (i, 0, 0)),
            pl.BlockSpec((P, H, W), lambda i: (i, 0, 0)),
            pl.BlockSpec((W, W), lambda i: (0, 0)),
            pl.BlockSpec((H, H), lambda i: (0, 0)),
        ],
        out_specs=pl.BlockSpec((1, 8, 128), lambda i: (i, 0, 0)),
        compiler_params=pltpu.CompilerParams(
            dimension_semantics=("parallel",)),
    )(x1, x2, aw, av)

    mean_ssim = jnp.sum(partials[:, 0, 0]) / float(nplanes * H * W)
    return 1.0 - mean_ssim


def kernel(img1, img2):
    return _ssim_loss(img1, img2)


# speedup vs baseline: 1.1481x; 1.0041x over previous
import functools

import numpy as np
import jax
import jax.numpy as jnp
from jax.experimental import pallas as pl
from jax.experimental.pallas import tpu as pltpu

_WINDOW = 11
_SIGMA = 1.5
_DATA_RANGE = 255.0
_K1 = 0.01
_K2 = 0.03


def _gauss_taps(window_size: int, sigma: float) -> np.ndarray:
    x = np.arange(window_size, dtype=np.float64) - window_size // 2
    g = np.exp(-(x * x) / (2.0 * sigma * sigma))
    return (g / g.sum()).astype(np.float32)


def _row_blur_matrix(n: int, taps: np.ndarray) -> np.ndarray:
    pad = taps.shape[0] // 2
    m = np.zeros((n, n), dtype=np.float32)
    for t, w in enumerate(taps):
        d = pad - t
        if abs(d) < n:
            m += w * np.eye(n, k=d, dtype=np.float32)
    return m


def _ssim_batch_kernel(x1_ref, x2_ref, aw_ref, av_ref, out_ref,
                       *, P, H, W, C1, C2):
    total = jnp.float32(0.0)
    for p in range(P):
        x1 = x1_ref[p].astype(jnp.bfloat16)
        x2 = x2_ref[p].astype(jnp.bfloat16)
        stack = jnp.concatenate(
            (x1, x2, x1 * x1 + x2 * x2, x1 * x2), axis=0)
        u = jnp.dot(stack, aw_ref[...],
                    preferred_element_type=jnp.float32).astype(jnp.bfloat16)
        blur = [jnp.dot(av_ref[...], u[s * H:(s + 1) * H],
                        preferred_element_type=jnp.float32).astype(jnp.bfloat16)
                for s in range(4)]
        mu1, mu2, e_ss, e12 = blur

        mu_ss = mu1 * mu1 + mu2 * mu2
        mu12 = mu1 * mu2
        num = (mu12 + mu12 + jnp.bfloat16(C1)) * \
            ((e12 - mu12) * jnp.bfloat16(2.0) + jnp.bfloat16(C2))
        den = (mu_ss + jnp.bfloat16(C1)) * ((e_ss - mu_ss) + jnp.bfloat16(C2))
        rs = jax.lax.rsqrt(den)
        total = total + jnp.sum(num * rs * rs, dtype=jnp.float32)
    out_ref[...] = jnp.full(out_ref.shape, total, out_ref.dtype)


def _ssim_loss(img1, img2):
    N, n_ch, H, W = img1.shape
    nplanes = N * n_ch

    P = next(p for p in (12, 8, 6, 4, 3, 2, 1) if nplanes % p == 0)
    S = nplanes // P

    taps = _gauss_taps(_WINDOW, _SIGMA)
    aw = jnp.asarray(_row_blur_matrix(W, taps)).astype(jnp.bfloat16)
    av = jnp.asarray(_row_blur_matrix(H, taps).T).astype(jnp.bfloat16)

    C1 = float((_K1 * _DATA_RANGE) ** 2)
    C2 = float((_K2 * _DATA_RANGE) ** 2)

    x1 = img1.astype(jnp.float32).reshape(nplanes, H, W)
    x2 = img2.astype(jnp.float32).reshape(nplanes, H, W)

    body = functools.partial(_ssim_batch_kernel, P=P, H=H, W=W, C1=C1, C2=C2)
    partials = pl.pallas_call(
        body,
        out_shape=jax.ShapeDtypeStruct((S, 8, 128), jnp.float32),
        grid=(S,),
        in_specs=[
            pl.BlockSpec((P, H, W), lambda i: (i, 0, 0)),
            pl.BlockSpec((P, H, W), lambda i: (i, 0, 0)),
            pl.BlockSpec((W, W), lambda i: (0, 0)),
            pl.BlockSpec((H, H), lambda i: (0, 0)),
        ],
        out_specs=pl.BlockSpec((1, 8, 128), lambda i: (i, 0, 0)),
        compiler_params=pltpu.CompilerParams(
            dimension_semantics=("parallel",)),
    )(x1, x2, aw, av)

    mean_ssim = jnp.sum(partials[:, 0, 0]) / float(nplanes * H * W)
    return 1.0 - mean_ssim


def kernel(img1, img2):
    return _ssim_loss(img1, img2)
